# baseline (speedup 1.0000x reference)
import functools

import jax
import jax.numpy as jnp
from jax import lax
from jax.experimental import pallas as pl
from jax.experimental.pallas import tpu as pltpu
from jax.experimental.pallas import tpu_sc as plsc

N = 1600000
B = 4096
Q = 16
WW = 32
SIGMA = 0.1
INV = 1.0 / (2.0 * SIGMA * SIGMA)
HI = lax.Precision.HIGHEST


C1 = 3200
NROW = 2 * N // 8


def _rep_body(d16_ref, txr_ref, tyr_ref, Pbot_ref, W2_ref, b2_ref,
              W3s_ref, b3_ref, R8_ref, out_ref):
    din = d16_ref[...].T
    x8 = din[0:8, :]
    y8 = din[8:16, :]
    ones = jnp.ones((1, C1), jnp.float32)
    inp = jnp.concatenate([din, ones], axis=0)
    mpre = jnp.dot(Pbot_ref[...], inp)
    xrep = jnp.broadcast_to(x8[:, None, :], (8, Q, C1)).reshape(8 * Q, C1)
    yrep = jnp.broadcast_to(y8[:, None, :], (8, Q, C1)).reshape(8 * Q, C1)
    ex = xrep - txr_ref[...]
    ey = yrep - tyr_ref[...]
    phi8 = jnp.exp((ex * ex + ey * ey) * (-INV))
    h1 = jnp.maximum(mpre, 0.0)
    h2 = jnp.dot(W2_ref[...], h1)
    h2 = jnp.maximum(h2 + b2_ref[...], 0.0)
    s8 = jnp.dot(W3s_ref[...], h2) + b3_ref[...]
    w8 = jnp.dot(R8_ref[...], jax.nn.sigmoid(s8))
    out_ref[...] = (phi8 * w8).T


def _rep_all(d16, txr, tyr, Pbot, W2, b2, W3s, b3, R8):
    grid = NROW // C1
    full = lambda a: pl.BlockSpec(a.shape, lambda i: (0,) * a.ndim)
    return pl.pallas_call(
        _rep_body,
        grid=(grid,),
        in_specs=[
            pl.BlockSpec((C1, 16), lambda i: (i, 0)),
            full(txr), full(tyr), full(Pbot), full(W2), full(b2), full(W3s),
            full(b3), full(R8),
        ],
        out_specs=pl.BlockSpec((C1, 128), lambda i: (i, 0)),
        out_shape=jax.ShapeDtypeStruct((NROW, 128), jnp.float32),
        compiler_params=pltpu.CompilerParams(
            dimension_semantics=("arbitrary",)),
    )(d16, txr, tyr, Pbot, W2, b2, W3s, b3, R8)


CH = 2560
ROWS = CH // 128
NCHUNK = N // CH
NSUB = 16
ROWS_PER_SUB = B // NSUB


def _seg_body(rep_hbm, ids_hbm, out_hbm, rep_v0, rep_v1, idx_v0, idx_v1,
              stage_v, acc_sh, gsem0, gsem1, ssem):
    c = lax.axis_index("c")
    s = lax.axis_index("s")

    def zrow(i, _):
        stage_v[i, :] = jnp.zeros((Q,), jnp.float32)
        return 0
    lax.fori_loop(0, ROWS_PER_SUB, zrow, 0)
    pltpu.sync_copy(stage_v, acc_sh.at[pl.ds(s * ROWS_PER_SUB, ROWS_PER_SUB)])
    plsc.subcore_barrier()

    base_cnt = NCHUNK // NSUB
    extra = NCHUNK - base_cnt * NSUB
    start = jnp.where(s == 0, 0, base_cnt * s + extra)
    stop = base_cnt * (s + 1) + extra

    def gather(k, rv, iv, sem):
        pltpu.async_copy(rep_hbm.at[pl.ds(c * N + k * CH, CH)], rv, sem)
        pltpu.async_copy(ids_hbm.at[pl.ds(c * N + k * CH, CH)], iv, sem)

    def gwait(k, rv, iv, sem):
        pltpu.make_async_copy(rep_hbm.at[pl.ds(c * N + k * CH, CH)], rv,
                              sem).wait()
        pltpu.make_async_copy(ids_hbm.at[pl.ds(c * N + k * CH, CH)], iv,
                              sem).wait()

    def scatter(rv, iv):
        ds = [pltpu.async_copy(rv.at[pl.ds(j * 128, 128)],
                               acc_sh.at[iv.at[pl.ds(j * 128, 128)]],
                               ssem, add=True)
              for j in range(ROWS)]
        for d in ds:
            d.wait()

    @pl.when(start < stop)
    def _():
        gather(start, rep_v0, idx_v0, gsem0)
    npairs = (stop - start + 1) // 2

    def pair(kk, _):
        k0 = start + 2 * kk
        k1 = k0 + 1
        gwait(k0, rep_v0, idx_v0, gsem0)

        @pl.when(k1 < stop)
        def _():
            gather(k1, rep_v1, idx_v1, gsem1)
        scatter(rep_v0, idx_v0)

        @pl.when(k1 < stop)
        def _():
            gwait(k1, rep_v1, idx_v1, gsem1)

            @pl.when(k1 + 1 < stop)
            def _():
                gather(k1 + 1, rep_v0, idx_v0, gsem0)
            scatter(rep_v1, idx_v1)
        return 0
    lax.fori_loop(0, npairs, pair, 0)

    plsc.subcore_barrier()
    pltpu.sync_copy(acc_sh.at[pl.ds(s * ROWS_PER_SUB, ROWS_PER_SUB)],
                    out_hbm.at[c].at[pl.ds(s * ROWS_PER_SUB, ROWS_PER_SUB)])


@functools.partial(
    pl.kernel,
    out_type=jax.ShapeDtypeStruct((2, B, Q), jnp.float32),
    mesh=plsc.VectorSubcoreMesh(core_axis_name="c", subcore_axis_name="s"),
    scratch_types=[
        pltpu.VMEM((CH, Q), jnp.float32),
        pltpu.VMEM((CH, Q), jnp.float32),
        pltpu.VMEM((CH,), jnp.int32),
        pltpu.VMEM((CH,), jnp.int32),
        pltpu.VMEM((ROWS_PER_SUB, Q), jnp.float32),
        pltpu.VMEM_SHARED((B, Q), jnp.float32),
        pltpu.SemaphoreType.DMA,
        pltpu.SemaphoreType.DMA,
        pltpu.SemaphoreType.DMA,
    ],
    compiler_params=pltpu.CompilerParams(use_tc_tiling_on_sc=False),
)
def _seg_kernel(rep_hbm, ids_hbm, out_hbm, rep_v0, rep_v1, idx_v0, idx_v1,
                stage_v, acc_sh, gsem0, gsem1, ssem):
    _seg_body(rep_hbm, ids_hbm, out_hbm, rep_v0, rep_v1, idx_v0, idx_v1,
              stage_v, acc_sh, gsem0, gsem1, ssem)


def _mlp_body(x_ref, gf_ref, r1a_ref, r1b_ref, r1c_ref, rb1_ref, rw2_ref,
              rb2_ref, rw3_ref, rb3_ref, out_ref):
    x0 = x_ref[0]
    x1 = x_ref[1]
    gf = gf_ref[...]
    h = jnp.dot(x0, r1a_ref[...])
    h += jnp.dot(x1, r1b_ref[...])
    h += jnp.dot(gf, r1c_ref[...])
    h = jnp.maximum(h + rb1_ref[...], 0.0)
    h2 = jnp.dot(h, rw2_ref[...])
    h2 = jnp.maximum(h2 + rb2_ref[...], 0.0)
    out = jnp.dot(h2, rw3_ref[...])
    out_ref[...] = out + rb3_ref[...]


def _mlp(x, gf, r1a, r1b, r1c, rb1, rw2, rb2, rw3, rb3):
    nc = rw3.shape[1]
    return pl.pallas_call(
        _mlp_body,
        out_shape=jax.ShapeDtypeStruct((B, nc), jnp.float32),
    )(x, gf, r1a, r1b, r1c, rb1, rw2, rb2, rw3, rb3)


@jax.jit
def kernel(diag0, diag1, graph_features, theta,
           ww1, wb1, ww2, wb2, ww3, wb3,
           rw1, rb1, rw2, rb2, rw3, rb3,
           batch0, batch1):
    f32 = jnp.float32
    d_all = jnp.concatenate([diag0, diag1], axis=0)
    d16 = jnp.swapaxes(d_all.reshape(NROW, 8, 2), 1, 2).reshape(NROW, 16)

    I8 = jnp.eye(8, dtype=f32)
    ones_q = jnp.ones((Q, 1), f32)
    txr = jnp.tile(theta[:, 0], 8)[:, None]
    tyr = jnp.tile(theta[:, 1], 8)[:, None]
    P1x = jnp.kron(I8, ww1[0][:, None])
    P1y = jnp.kron(I8, ww1[1][:, None])
    b1_8 = jnp.tile(wb1, 8)[:, None]
    Pbot = jnp.concatenate([P1x, P1y, b1_8], axis=1)
    W2 = jnp.kron(I8, ww2.T)
    b2 = jnp.tile(wb2, 8)[:, None]
    W3s = jnp.kron(I8, ww3.T)
    R8 = jnp.kron(I8, ones_q)

    rep = _rep_all(d16, txr, tyr, Pbot, W2, b2, W3s, wb3[0].reshape(1, 1),
                   R8)

    ids2 = jnp.concatenate([batch0, batch1]).astype(jnp.int32)
    x = _seg_kernel(rep.reshape(2 * N, Q), ids2)

    return _mlp(x, graph_features, rw1[:Q, :], rw1[Q:2 * Q, :], rw1[2 * Q:, :],
                rb1[None, :], rw2, rb2[None, :], rw3, rb3[None, :])


# speedup vs baseline: 2.1237x; 2.1237x over previous
import functools

import jax
import jax.numpy as jnp
from jax import lax
from jax.experimental import pallas as pl
from jax.experimental.pallas import tpu as pltpu
from jax.experimental.pallas import tpu_sc as plsc

N = 1600000
B = 4096
Q = 16
WW = 32
SIGMA = 0.1
INV = 1.0 / (2.0 * SIGMA * SIGMA)
HI = lax.Precision.HIGHEST


C1 = 3200
NROW = 2 * N // 8


def _rep_body(dstack_ref, txr_ref, tyr_ref, Pbot_ref, W2_ref, b2_ref,
              W3s_ref, b3_ref, R8_ref, out_ref):
    din = dstack_ref[...]
    x8 = din[0:8, :]
    y8 = din[8:16, :]
    ones = jnp.ones((1, C1), jnp.float32)
    inp = jnp.concatenate([din, ones], axis=0)
    mpre = jnp.dot(Pbot_ref[...], inp)
    xrep = jnp.broadcast_to(x8[:, None, :], (8, Q, C1)).reshape(8 * Q, C1)
    yrep = jnp.broadcast_to(y8[:, None, :], (8, Q, C1)).reshape(8 * Q, C1)
    ex = xrep - txr_ref[...]
    ey = yrep - tyr_ref[...]
    phi8 = jnp.exp((ex * ex + ey * ey) * (-INV))
    h1 = jnp.maximum(mpre, 0.0)
    h2 = jnp.dot(W2_ref[...], h1)
    h2 = jnp.maximum(h2 + b2_ref[...], 0.0)
    s8 = jnp.dot(W3s_ref[...], h2) + b3_ref[...]
    w8 = jnp.dot(R8_ref[...], jax.nn.sigmoid(s8))
    out_ref[...] = (phi8 * w8).T


def _rep_all(dstack, txr, tyr, Pbot, W2, b2, W3s, b3, R8):
    grid = pl.cdiv(N // 8, C1)
    full = lambda a: pl.BlockSpec(a.shape, lambda i: (0,) * a.ndim)
    return pl.pallas_call(
        _rep_body,
        grid=(grid,),
        in_specs=[
            pl.BlockSpec((16, C1), lambda i: (0, i)),
            full(txr), full(tyr), full(Pbot), full(W2), full(b2), full(W3s),
            full(b3), full(R8),
        ],
        out_specs=pl.BlockSpec((C1, 128), lambda i: (i, 0)),
        out_shape=jax.ShapeDtypeStruct((N // 8, 128), jnp.float32),
        compiler_params=pltpu.CompilerParams(
            dimension_semantics=("arbitrary",)),
    )(dstack, txr, tyr, Pbot, W2, b2, W3s, b3, R8)


CH = 2560
ROWS = CH // 128
NCHUNK = N // CH
CORE0_CHUNKS = 320
NSUB = 16
ROWS_PER_SUB = B // NSUB


def _seg_body(rep_hbm, ids_hbm, out_hbm, rep_v0, rep_v1, idx_v0, idx_v1,
              stage_v, acc_sh, gsem0, gsem1, ssem):
    c = lax.axis_index("c")
    s = lax.axis_index("s")

    def zrow(i, _):
        stage_v[i, :] = jnp.zeros((Q,), jnp.float32)
        return 0
    lax.fori_loop(0, ROWS_PER_SUB, zrow, 0)
    pltpu.sync_copy(stage_v, acc_sh.at[pl.ds(s * ROWS_PER_SUB, ROWS_PER_SUB)])
    plsc.subcore_barrier()

    core_lo = jnp.where(c == 0, 0, CORE0_CHUNKS)
    ncc = jnp.where(c == 0, CORE0_CHUNKS, NCHUNK - CORE0_CHUNKS)
    base_cnt = ncc // NSUB
    extra = ncc - base_cnt * NSUB
    start = core_lo + jnp.where(s == 0, 0, base_cnt * s + extra)
    stop = core_lo + base_cnt * (s + 1) + extra

    def gather(k, rv, iv, sem):
        pltpu.async_copy(rep_hbm.at[pl.ds(k * CH, CH)], rv, sem)
        pltpu.async_copy(ids_hbm.at[pl.ds(k * CH, CH)], iv, sem)

    def gwait(k, rv, iv, sem):
        pltpu.make_async_copy(rep_hbm.at[pl.ds(k * CH, CH)], rv, sem).wait()
        pltpu.make_async_copy(ids_hbm.at[pl.ds(k * CH, CH)], iv, sem).wait()

    def scatter(rv, iv):
        ds = [pltpu.async_copy(rv.at[pl.ds(j * 128, 128)],
                               acc_sh.at[iv.at[pl.ds(j * 128, 128)]],
                               ssem, add=True)
              for j in range(ROWS)]
        for d in ds:
            d.wait()

    @pl.when(start < stop)
    def _():
        gather(start, rep_v0, idx_v0, gsem0)
    npairs = (stop - start + 1) // 2

    def pair(kk, _):
        k0 = start + 2 * kk
        k1 = k0 + 1
        gwait(k0, rep_v0, idx_v0, gsem0)

        @pl.when(k1 < stop)
        def _():
            gather(k1, rep_v1, idx_v1, gsem1)
        scatter(rep_v0, idx_v0)

        @pl.when(k1 < stop)
        def _():
            gwait(k1, rep_v1, idx_v1, gsem1)

            @pl.when(k1 + 1 < stop)
            def _():
                gather(k1 + 1, rep_v0, idx_v0, gsem0)
            scatter(rep_v1, idx_v1)
        return 0
    lax.fori_loop(0, npairs, pair, 0)

    plsc.subcore_barrier()
    pltpu.sync_copy(acc_sh.at[pl.ds(s * ROWS_PER_SUB, ROWS_PER_SUB)],
                    out_hbm.at[c].at[pl.ds(s * ROWS_PER_SUB, ROWS_PER_SUB)])


@functools.partial(
    pl.kernel,
    out_type=jax.ShapeDtypeStruct((2, B, Q), jnp.float32),
    mesh=plsc.VectorSubcoreMesh(core_axis_name="c", subcore_axis_name="s"),
    scratch_types=[
        pltpu.VMEM((CH, Q), jnp.float32),
        pltpu.VMEM((CH, Q), jnp.float32),
        pltpu.VMEM((CH,), jnp.int32),
        pltpu.VMEM((CH,), jnp.int32),
        pltpu.VMEM((ROWS_PER_SUB, Q), jnp.float32),
        pltpu.VMEM_SHARED((B, Q), jnp.float32),
        pltpu.SemaphoreType.DMA,
        pltpu.SemaphoreType.DMA,
        pltpu.SemaphoreType.DMA,
    ],
    compiler_params=pltpu.CompilerParams(use_tc_tiling_on_sc=False),
)
def _seg_kernel(rep_hbm, ids_hbm, out_hbm, rep_v0, rep_v1, idx_v0, idx_v1,
                stage_v, acc_sh, gsem0, gsem1, ssem):
    _seg_body(rep_hbm, ids_hbm, out_hbm, rep_v0, rep_v1, idx_v0, idx_v1,
              stage_v, acc_sh, gsem0, gsem1, ssem)


def _mlp_body(xa_ref, xb_ref, gf_ref, r1a_ref, r1b_ref, r1c_ref, rb1_ref,
              rw2_ref, rb2_ref, rw3_ref, rb3_ref, out_ref):
    x0 = xa_ref[0] + xa_ref[1]
    x1 = xb_ref[0] + xb_ref[1]
    gf = gf_ref[...]
    h = jnp.dot(x0, r1a_ref[...])
    h += jnp.dot(x1, r1b_ref[...])
    h += jnp.dot(gf, r1c_ref[...])
    h = jnp.maximum(h + rb1_ref[...], 0.0)
    h2 = jnp.dot(h, rw2_ref[...])
    h2 = jnp.maximum(h2 + rb2_ref[...], 0.0)
    out = jnp.dot(h2, rw3_ref[...])
    out_ref[...] = out + rb3_ref[...]


def _mlp(xa, xb, gf, r1a, r1b, r1c, rb1, rw2, rb2, rw3, rb3):
    nc = rw3.shape[1]
    return pl.pallas_call(
        _mlp_body,
        out_shape=jax.ShapeDtypeStruct((B, nc), jnp.float32),
    )(xa, xb, gf, r1a, r1b, r1c, rb1, rw2, rb2, rw3, rb3)


@jax.jit
def kernel(diag0, diag1, graph_features, theta,
           ww1, wb1, ww2, wb2, ww3, wb3,
           rw1, rb1, rw2, rb2, rw3, rb3,
           batch0, batch1):
    f32 = jnp.float32
    def mk_dstack(dg):
        xr = dg[:, 0].reshape(N // 8, 8).T
        yr = dg[:, 1].reshape(N // 8, 8).T
        return jnp.concatenate([xr, yr], axis=0)

    I8 = jnp.eye(8, dtype=f32)
    ones_q = jnp.ones((Q, 1), f32)
    txr = jnp.tile(theta[:, 0], 8)[:, None]
    tyr = jnp.tile(theta[:, 1], 8)[:, None]
    P1x = jnp.kron(I8, ww1[0][:, None])
    P1y = jnp.kron(I8, ww1[1][:, None])
    b1_8 = jnp.tile(wb1, 8)[:, None]
    Pbot = jnp.concatenate([P1x, P1y, b1_8], axis=1)
    W2 = jnp.kron(I8, ww2.T)
    b2 = jnp.tile(wb2, 8)[:, None]
    W3s = jnp.kron(I8, ww3.T)
    R8 = jnp.kron(I8, ones_q)

    b3r = wb3[0].reshape(1, 1)
    repa = _rep_all(mk_dstack(diag0), txr, tyr, Pbot, W2, b2, W3s, b3r, R8)
    xa = _seg_kernel(repa.reshape(N, Q), batch0.astype(jnp.int32))
    repb = _rep_all(mk_dstack(diag1), txr, tyr, Pbot, W2, b2, W3s, b3r, R8)
    xb = _seg_kernel(repb.reshape(N, Q), batch1.astype(jnp.int32))

    return _mlp(xa, xb, graph_features, rw1[:Q, :], rw1[Q:2 * Q, :],
                rw1[2 * Q:, :], rb1[None, :], rw2, rb2[None, :], rw3,
                rb3[None, :])
